# baseline (speedup 1.0000x reference)
import functools

import jax
import jax.numpy as jnp
from jax import lax
from jax.experimental import pallas as pl
from jax.experimental.pallas import tpu as pltpu
from jax.experimental.pallas import tpu_sc as plsc

K = 8
B, D, N = 4, 2048, 8192
R = B * D
L = 16
VPR = N // L
NC, NS = 2, 16
NW = NC * NS
RPW = R // NW
CR = 4
CW = CR * N
NCH = RPW // CR
NSLOT = N + 32
NEG = float("-inf")
DELTA = 0.25


def _sortd(v):
    sk, _ = plsc.sort_key_val(v, v, descending=True)
    return sk


def _msort(a, b, lane):
    comb = jnp.where(lane < K, a, lax.rev(b, (0,)))
    return _sortd(comb)


def _scalar0(v):
    return lax.squeeze(lax.slice(v, (0,), (1,)), (0,))


def _sc_body(x_hbm, out_hbm, buf, slots, stage, nslot, test_ref, sem0, sem1):
    cid = lax.axis_index("c")
    sid = lax.axis_index("s")
    w = sid * NC + cid
    row0 = w * RPW
    base_off = row0 * N
    b_idx = w // (D // RPW)
    d0 = (w % (D // RPW)) * RPW

    lane = lax.iota(jnp.int32, L)
    lt8 = lane < K
    neg_v = jnp.full((L,), NEG, jnp.float32)

    def copy(c, par, sem):
        return pltpu.make_async_copy(
            x_hbm.at[pl.ds(base_off + c * CW, CW)],
            buf.at[pl.ds(par * CW, CW)],
            sem,
        )

    copy(0, 0, sem0).start()
    copy(1, 1, sem1).start()
    test_ref[0] = jnp.float32(jnp.inf)

    def row_scan(rb, t_s):
        t_vec = jnp.full((L,), t_s, jnp.float32)

        def body(i, carry):
            m_run, ptr = carry
            x = buf[pl.ds(rb + i * L, L)]
            m_run = jnp.maximum(m_run, x)
            mask = x >= t_vec
            plsc.store_compressed(slots.at[pl.ds(ptr, L)], x, mask=mask)
            cnt = plsc.all_reduce_population_count(mask)
            return m_run, ptr + _scalar0(cnt)

        return lax.fori_loop(0, VPR, body, (neg_v, jnp.int32(0)), unroll=8)

    def chunk_body(c, carry):
        par = c & 1
        pbase = par * CW

        @pl.when(par == 0)
        def _():
            copy(c, 0, sem0).wait()

        @pl.when(par == 1)
        def _():
            copy(c, 1, sem1).wait()

        def row_body(r, _):
            rb = pbase + r * N
            t_est = test_ref[0]
            m_run, ptr = row_scan(rb, t_est)

            sm = _sortd(m_run)
            t_ex = jnp.max(jnp.where(lane == K - 1, sm, NEG))
            test_ref[0] = t_ex - jnp.float32(DELTA)
            nslot[0] = ptr

            @pl.when(t_est > t_ex)
            def _():
                _, p2 = row_scan(rb, t_ex)
                nslot[0] = p2

            ptr_f = nslot[0]
            slots[pl.ds(ptr_f, L)] = neg_v
            nq = (ptr_f + 15) >> 4

            def tmerge(ci, acc):
                s = _sortd(slots[pl.ds(ci * L, L)])
                return _msort(acc, s, lane)

            top8 = lax.fori_loop(0, nq, tmerge, neg_v)

            i_row = c * CR + r
            plsc.store_scatter(stage, [lane * RPW + i_row], top8, mask=lt8)
            return 0

        lax.fori_loop(0, CR, row_body, 0)

        c2 = c + 2

        @pl.when((c2 < NCH) & (par == 0))
        def _():
            copy(c2, 0, sem0).start()

        @pl.when((c2 < NCH) & (par == 1))
        def _():
            copy(c2, 1, sem1).start()

        return carry

    lax.fori_loop(0, NCH, chunk_body, 0)

    for j in range(K):
        pltpu.sync_copy(
            stage.at[pl.ds(j * RPW, RPW)],
            out_hbm.at[b_idx, j, pl.ds(d0, RPW)],
        )


@functools.partial(
    pl.kernel,
    out_type=jax.ShapeDtypeStruct((B, K, D), jnp.float32),
    mesh=plsc.VectorSubcoreMesh(core_axis_name="c", subcore_axis_name="s"),
    compiler_params=pltpu.CompilerParams(needs_layout_passes=False),
    scratch_types=[
        pltpu.VMEM((2 * CW,), jnp.float32),
        pltpu.VMEM((NSLOT,), jnp.float32),
        pltpu.VMEM((K * RPW,), jnp.float32),
        pltpu.SMEM((1,), jnp.int32),
        pltpu.SMEM((1,), jnp.float32),
        pltpu.SemaphoreType.DMA,
        pltpu.SemaphoreType.DMA,
    ],
)
def _sc_topk(x_hbm, out_hbm, buf, slots, stage, nslot, test_ref, sem0, sem1):
    _sc_body(x_hbm, out_hbm, buf, slots, stage, nslot, test_ref, sem0, sem1)


def kernel(inputs):
    return _sc_topk(inputs.reshape(-1))
